# Initial kernel scaffold; baseline (speedup 1.0000x reference)
import jax
import jax.numpy as jnp
from jax.experimental import pallas as pl


def kernel(feats, edge_index, seq_W, seq_b, W0, b0, W1, b1, W2, b2, ln_s0, ln_b0, ln_s1, ln_b1, ln_s2, ln_b2, cls_W, cls_b):
    raise NotImplementedError("write your pallas kernel here")


# speedup vs baseline: 7.1981x; 7.1981x over previous
import functools

import jax
import jax.numpy as jnp
from jax import lax
from jax.experimental import pallas as pl
from jax.experimental.pallas import tpu as pltpu
from jax.experimental.pallas import tpu_sc as plsc

N = 50000
E = 800000
R = 64
L = 50
NCLS = 10
EROWS = E // 128
NSC = 2
NTILE = 16
KB = 8
KA = 4
CHUNKS_PT = 392
EROWS_P = CHUNKS_PT * NTILE
NP = 50048
ROWS_PT = NP // NTILE

_mesh = plsc.VectorSubcoreMesh(
    core_axis_name="c", subcore_axis_name="s", num_cores=NSC,
    num_subcores=NTILE)

_sc_params = pltpu.CompilerParams(use_tc_tiling_on_sc=False)


@functools.partial(
    pl.kernel,
    out_type=jax.ShapeDtypeStruct((2 * NP, 16), jnp.float32),
    mesh=_mesh,
    scratch_types=[
        pltpu.VMEM((KB, 128), jnp.int32),
        pltpu.VMEM((128, 16), jnp.float32),
        pltpu.VMEM_SHARED((NP, 16), jnp.float32),
    ],
    compiler_params=_sc_params,
)
def _degrees(hidx_hbm, ones_hbm, zeros_hbm, cnt_hbm, idx_v, ones_v, cnt_sh):
  c = lax.axis_index("c")
  s = lax.axis_index("s")
  pltpu.sync_copy(ones_hbm, ones_v)
  pltpu.sync_copy(zeros_hbm, cnt_sh.at[pl.ds(s * ROWS_PT, ROWS_PT)])
  plsc.subcore_barrier()

  start = s * CHUNKS_PT

  def grp_body(g, carry):
    r0 = start + g * KB
    pltpu.sync_copy(hidx_hbm.at[c, pl.ds(r0, KB)], idx_v)
    for j in range(KB):
      pltpu.sync_copy(ones_v, cnt_sh.at[idx_v.at[j]], add=True)
    return carry

  lax.fori_loop(0, CHUNKS_PT // KB, grp_body, 0)
  plsc.subcore_barrier()
  pltpu.sync_copy(cnt_sh.at[pl.ds(s * ROWS_PT, ROWS_PT)],
                  cnt_hbm.at[pl.ds(c * NP + s * ROWS_PT, ROWS_PT)])


@functools.partial(
    pl.kernel,
    out_type=jax.ShapeDtypeStruct((2 * NP, 32), jnp.float32),
    mesh=_mesh,
    scratch_types=[
        pltpu.VMEM((KA, 128), jnp.int32),
        pltpu.VMEM((KA, 128), jnp.int32),
        pltpu.VMEM((KA, 128, 32), jnp.float32),
        pltpu.VMEM_SHARED((NP, 32), jnp.float32),
        pltpu.SemaphoreType.DMA,
    ],
    compiler_params=_sc_params,
)
def _edge_agg(ycat_hbm, src2_hbm, dstr_hbm, zeros_hbm, agg_hbm, sidx, didx,
              rows, agg_sh, sem):
  c = lax.axis_index("c")
  s = lax.axis_index("s")
  pltpu.sync_copy(zeros_hbm, agg_sh.at[pl.ds(s * ROWS_PT, ROWS_PT)])
  plsc.subcore_barrier()

  start = s * CHUNKS_PT

  def grp_body(g, carry):
    r0 = start + g * KA
    pltpu.sync_copy(src2_hbm.at[c, pl.ds(r0, KA)], sidx)
    pltpu.sync_copy(dstr_hbm.at[pl.ds(r0, KA)], didx)
    cps = [
        pltpu.async_copy(ycat_hbm.at[sidx.at[j]], rows.at[j], sem)
        for j in range(KA)
    ]
    for cp in cps:
      cp.wait()
    for j in range(KA):
      pltpu.sync_copy(rows.at[j], agg_sh.at[didx.at[j]], add=True)
    return carry

  lax.fori_loop(0, CHUNKS_PT // KA, grp_body, 0)
  plsc.subcore_barrier()
  pltpu.sync_copy(agg_sh.at[pl.ds(s * ROWS_PT, ROWS_PT)],
                  agg_hbm.at[pl.ds(c * NP + s * ROWS_PT, ROWS_PT)])


_BN = 2000


def _norms(deg_blk):
  ns = lax.rsqrt(jnp.maximum(deg_blk[0][:, :1], 1.0))
  nd = lax.rsqrt(jnp.maximum(deg_blk[1][:, :1], 1.0))
  return ns, nd


def _k1_body(f_ref, deg_ref, sqW_ref, sqb_ref, W_ref, y_ref):
  x = f_ref[...] * sqW_ref[...] + sqb_ref[...]
  ns, _ = _norms(deg_ref[...])
  y = jnp.dot(x * ns, W_ref[...], preferred_element_type=jnp.float32)
  y_ref[0] = y[:, :32]
  y_ref[1] = y[:, 32:]


def _epilogue(agg_blk, nd, b, ls, lb):
  a = jnp.concatenate([agg_blk[0], agg_blk[1]], axis=-1)
  t = a * nd + b
  mu = jnp.mean(t, axis=-1, keepdims=True)
  var = jnp.mean(jnp.square(t - mu), axis=-1, keepdims=True)
  t = (t - mu) * lax.rsqrt(var + 1e-5) * ls + lb
  return jnp.maximum(t, 0.0)


def _k2_body(agg_ref, deg_ref, b_ref, ls_ref, lb_ref, W_ref, y_ref):
  ns, nd = _norms(deg_ref[...])
  x = _epilogue(agg_ref[...], nd, b_ref[...], ls_ref[...], lb_ref[...])
  y = jnp.dot(x * ns, W_ref[...], preferred_element_type=jnp.float32)
  y_ref[0] = y[:, :32]
  y_ref[1] = y[:, 32:]


def _k3_body(agg_ref, deg_ref, b_ref, ls_ref, lb_ref, x_ref):
  _, nd = _norms(deg_ref[...])
  x_ref[...] = _epilogue(agg_ref[...], nd, b_ref[...], ls_ref[...],
                         lb_ref[...])


def _k4_body(x_ref, W_ref, b_ref, y_ref):
  y_ref[...] = (
      jnp.dot(x_ref[...], W_ref[...], preferred_element_type=jnp.float32)
      + b_ref[...])


def _full(shape):
  return pl.BlockSpec(shape, lambda i: tuple(0 for _ in shape))


def _rows3(shape):
  return pl.BlockSpec(shape, lambda i: (0, i, 0))


def _layer_matmul_first(feats_col, deg2, seq_W, seq_b, W):
  return pl.pallas_call(
      _k1_body,
      grid=(N // _BN,),
      in_specs=[
          pl.BlockSpec((_BN, 1), lambda i: (i, 0)),
          _rows3((2, _BN, 16)),
          _full((1, R)),
          _full((1, R)),
          _full((R, R)),
      ],
      out_specs=_rows3((2, _BN, 32)),
      out_shape=jax.ShapeDtypeStruct((2, N, 32), jnp.float32),
  )(feats_col, deg2, seq_W, seq_b, W)


def _layer_matmul_mid(agg2, deg2, b, ls, lb, W):
  return pl.pallas_call(
      _k2_body,
      grid=(N // _BN,),
      in_specs=[
          _rows3((2, _BN, 32)),
          _rows3((2, _BN, 16)),
          _full((1, R)),
          _full((1, R)),
          _full((1, R)),
          _full((R, R)),
      ],
      out_specs=_rows3((2, _BN, 32)),
      out_shape=jax.ShapeDtypeStruct((2, N, 32), jnp.float32),
  )(agg2, deg2, b, ls, lb, W)


def _final_epilogue(agg2, deg2, b, ls, lb):
  return pl.pallas_call(
      _k3_body,
      grid=(N // _BN,),
      in_specs=[
          _rows3((2, _BN, 32)),
          _rows3((2, _BN, 16)),
          _full((1, R)),
          _full((1, R)),
          _full((1, R)),
      ],
      out_specs=pl.BlockSpec((_BN, R), lambda i: (i, 0)),
      out_shape=jax.ShapeDtypeStruct((N, R), jnp.float32),
  )(agg2, deg2, b, ls, lb)


def _classifier(xr, cls_W, cls_b):
  bs = N // L
  br = 200
  return pl.pallas_call(
      _k4_body,
      grid=(bs // br,),
      in_specs=[
          pl.BlockSpec((br, R * L), lambda i: (i, 0)),
          _full((R * L, NCLS)),
          _full((1, NCLS)),
      ],
      out_specs=pl.BlockSpec((br, NCLS), lambda i: (i, 0)),
      out_shape=jax.ShapeDtypeStruct((bs, NCLS), jnp.float32),
  )(xr, cls_W, cls_b)


def kernel(feats, edge_index, seq_W, seq_b, W0, b0, W1, b1, W2, b2,
           ln_s0, ln_b0, ln_s1, ln_b1, ln_s2, ln_b2, cls_W, cls_b):
  src = edge_index[0]
  dst = edge_index[1]
  srcr = src.reshape(EROWS, 128)
  dstr = dst.reshape(EROWS, 128)
  npad = EROWS_P - EROWS
  lanes = lax.broadcasted_iota(jnp.int32, (npad, 128), 1)
  pad_src = lanes
  pad_trash = N + (lanes % (NP - N))
  srcr_p = jnp.concatenate([srcr, pad_src], axis=0)
  dstr_p = jnp.concatenate([dstr, pad_trash], axis=0)
  src2 = jnp.stack([srcr_p, srcr_p + N])
  hidx = jnp.stack([jnp.concatenate([srcr, pad_trash], axis=0), dstr_p])
  ones16 = jnp.ones((128, 16), jnp.float32)
  zeros16 = jnp.zeros((ROWS_PT, 16), jnp.float32)
  zeros32 = jnp.zeros((ROWS_PT, 32), jnp.float32)

  cnt = _degrees(hidx, ones16, zeros16)
  deg2 = cnt.reshape(2, NP, 16)

  feats_col = feats.reshape(N, 1)
  row = lambda v: v.reshape(1, R)

  y = _layer_matmul_first(feats_col, deg2, seq_W, row(seq_b), W0)
  agg2 = _edge_agg(y.reshape(2 * N, 32), src2, dstr_p,
                   zeros32).reshape(2, NP, 32)

  for (b, ls, lb, Wn) in ((b0, ln_s0, ln_b0, W1), (b1, ln_s1, ln_b1, W2)):
    y = _layer_matmul_mid(agg2, deg2, row(b), row(ls), row(lb), Wn)
    agg2 = _edge_agg(y.reshape(2 * N, 32), src2, dstr_p,
                     zeros32).reshape(2, NP, 32)

  x = _final_epilogue(agg2, deg2, row(b2), row(ln_s2), row(ln_b2))
  xr = x.reshape(N // L, R * L)
  return _classifier(xr, cls_W, jnp.reshape(cls_b, (1, NCLS)))


# speedup vs baseline: 12.6528x; 1.7578x over previous
import functools

import jax
import jax.numpy as jnp
from jax import lax
from jax.experimental import pallas as pl
from jax.experimental.pallas import tpu as pltpu
from jax.experimental.pallas import tpu_sc as plsc

N = 50000
E = 800000
R = 64
L = 50
NCLS = 10
EROWS = E // 128
NSC = 2
NTILE = 16
SUP = 40
NSUP = 10
CHUNKS_PT = SUP * NSUP
EROWS_P = CHUNKS_PT * NTILE
NBUF = 5
NP = 50048
ROWS_PT = NP // NTILE

_mesh = plsc.VectorSubcoreMesh(
    core_axis_name="c", subcore_axis_name="s", num_cores=NSC,
    num_subcores=NTILE)

_sc_params = pltpu.CompilerParams(use_tc_tiling_on_sc=False)


@functools.partial(
    pl.kernel,
    out_type=jax.ShapeDtypeStruct((2 * NP, 16), jnp.float32),
    mesh=_mesh,
    scratch_types=[
        pltpu.VMEM((SUP, 128), jnp.int32),
        pltpu.VMEM((128, 16), jnp.float32),
        pltpu.VMEM_SHARED((NP, 16), jnp.float32),
        pltpu.SemaphoreType.DMA,
    ],
    compiler_params=_sc_params,
)
def _degrees(hidx_hbm, ones_hbm, zeros_hbm, cnt_hbm, idx_v, ones_v, cnt_sh,
             sem):
  c = lax.axis_index("c")
  s = lax.axis_index("s")
  pltpu.sync_copy(ones_hbm, ones_v)
  pltpu.sync_copy(zeros_hbm, cnt_sh.at[pl.ds(s * ROWS_PT, ROWS_PT)])
  plsc.subcore_barrier()

  start = s * CHUNKS_PT

  def sup_body(u, carry):
    r0 = start + u * SUP
    pltpu.sync_copy(hidx_hbm.at[c, pl.ds(r0, SUP)], idx_v)

    def fire(i, carry2):
      for b in range(4):
        pltpu.async_copy(ones_v, cnt_sh.at[idx_v.at[i * 4 + b]], sem,
                         add=True)
      return carry2

    lax.fori_loop(0, SUP // 4, fire, 0)

    def drain(i, carry2):
      for b in range(4):
        pltpu.make_async_copy(ones_v, cnt_sh.at[idx_v.at[i * 4 + b]],
                              sem).wait()
      return carry2

    lax.fori_loop(0, SUP // 4, drain, 0)
    return carry

  lax.fori_loop(0, NSUP, sup_body, 0)
  plsc.subcore_barrier()
  pltpu.sync_copy(cnt_sh.at[pl.ds(s * ROWS_PT, ROWS_PT)],
                  cnt_hbm.at[pl.ds(c * NP + s * ROWS_PT, ROWS_PT)])


@functools.partial(
    pl.kernel,
    out_type=jax.ShapeDtypeStruct((2 * NP, 32), jnp.float32),
    mesh=_mesh,
    scratch_types=[
        pltpu.VMEM((SUP, 128), jnp.int32),
        pltpu.VMEM((SUP, 128), jnp.int32),
        [pltpu.VMEM((128, 32), jnp.float32)] * NBUF,
        pltpu.VMEM_SHARED((NP, 32), jnp.float32),
        [pltpu.SemaphoreType.DMA] * NBUF,
        [pltpu.SemaphoreType.DMA] * NBUF,
    ],
    compiler_params=_sc_params,
)
def _edge_agg(ycat_hbm, src2_hbm, dstr_hbm, zeros_hbm, agg_hbm, sidx, didx,
              rows, agg_sh, gsem, ssem):
  c = lax.axis_index("c")
  s = lax.axis_index("s")
  pltpu.sync_copy(zeros_hbm, agg_sh.at[pl.ds(s * ROWS_PT, ROWS_PT)])
  plsc.subcore_barrier()

  start = s * CHUNKS_PT

  def gath(b, g):
    pltpu.async_copy(ycat_hbm.at[sidx.at[g]], rows[b], gsem[b])

  def wait_gath(b):
    pltpu.make_async_copy(ycat_hbm.at[sidx.at[0]], rows[b], gsem[b]).wait()

  def scat(b, g):
    pltpu.async_copy(rows[b], agg_sh.at[didx.at[g]], ssem[b], add=True)

  def wait_scat(b):
    pltpu.make_async_copy(rows[b], agg_sh.at[didx.at[0]], ssem[b]).wait()

  def sup_body(u, carry):
    r0 = start + u * SUP
    pltpu.sync_copy(src2_hbm.at[c, pl.ds(r0, SUP)], sidx)
    pltpu.sync_copy(dstr_hbm.at[pl.ds(r0, SUP)], didx)
    for b in range(NBUF):
      gath(b, b)

    def stage(i, carry2):
      for b in range(NBUF):
        wait_gath(b)
        scat(b, NBUF * i + b)
      for b in range(NBUF):
        wait_scat(b)
        gath(b, NBUF * (i + 1) + b)
      return carry2

    lax.fori_loop(0, SUP // NBUF - 1, stage, 0)
    for b in range(NBUF):
      wait_gath(b)
      scat(b, SUP - NBUF + b)
    for b in range(NBUF):
      wait_scat(b)
    return carry

  lax.fori_loop(0, NSUP, sup_body, 0)
  plsc.subcore_barrier()
  pltpu.sync_copy(agg_sh.at[pl.ds(s * ROWS_PT, ROWS_PT)],
                  agg_hbm.at[pl.ds(c * NP + s * ROWS_PT, ROWS_PT)])


NSUP16 = EROWS_P // (2 * NTILE * SUP)


@functools.partial(
    pl.kernel,
    out_type=jax.ShapeDtypeStruct((2 * NP, 16), jnp.float32),
    mesh=_mesh,
    scratch_types=[
        pltpu.VMEM((SUP, 128), jnp.int32),
        pltpu.VMEM((SUP, 128), jnp.int32),
        [pltpu.VMEM((128, 16), jnp.float32)] * NBUF,
        pltpu.VMEM_SHARED((NP, 16), jnp.float32),
        [pltpu.SemaphoreType.DMA] * NBUF,
        [pltpu.SemaphoreType.DMA] * NBUF,
    ],
    compiler_params=_sc_params,
)
def _edge_agg16(g1_hbm, src2_hbm, dstr_hbm, zeros_hbm, agg_hbm, sidx, didx,
                rows, agg_sh, gsem, ssem):
  c = lax.axis_index("c")
  s = lax.axis_index("s")
  pltpu.sync_copy(zeros_hbm, agg_sh.at[pl.ds(s * ROWS_PT, ROWS_PT)])
  plsc.subcore_barrier()

  start = (c * NTILE + s) * (NSUP16 * SUP)

  def gath(b, g):
    pltpu.async_copy(g1_hbm.at[sidx.at[g]], rows[b], gsem[b])

  def wait_gath(b):
    pltpu.make_async_copy(g1_hbm.at[sidx.at[0]], rows[b], gsem[b]).wait()

  def scat(b, g):
    pltpu.async_copy(rows[b], agg_sh.at[didx.at[g]], ssem[b], add=True)

  def wait_scat(b):
    pltpu.make_async_copy(rows[b], agg_sh.at[didx.at[0]], ssem[b]).wait()

  def sup_body(u, carry):
    r0 = start + u * SUP
    pltpu.sync_copy(src2_hbm.at[0, pl.ds(r0, SUP)], sidx)
    pltpu.sync_copy(dstr_hbm.at[pl.ds(r0, SUP)], didx)
    for b in range(NBUF):
      gath(b, b)

    def stage(i, carry2):
      for b in range(NBUF):
        wait_gath(b)
        scat(b, NBUF * i + b)
      for b in range(NBUF):
        wait_scat(b)
        gath(b, NBUF * (i + 1) + b)
      return carry2

    lax.fori_loop(0, SUP // NBUF - 1, stage, 0)
    for b in range(NBUF):
      wait_gath(b)
      scat(b, SUP - NBUF + b)
    for b in range(NBUF):
      wait_scat(b)
    return carry

  lax.fori_loop(0, NSUP16, sup_body, 0)
  plsc.subcore_barrier()
  pltpu.sync_copy(agg_sh.at[pl.ds(s * ROWS_PT, ROWS_PT)],
                  agg_hbm.at[pl.ds(c * NP + s * ROWS_PT, ROWS_PT)])


_BN = 5000


def _norms(deg_blk):
  ns = lax.rsqrt(jnp.maximum(deg_blk[0][:, :1], 1.0))
  nd = lax.rsqrt(jnp.maximum(deg_blk[1][:, :1], 1.0))
  return ns, nd


def _g1_body(f_ref, deg_ref, g_ref):
  ns, _ = _norms(deg_ref[...])
  nsf = ns * f_ref[...]
  lane = lax.broadcasted_iota(jnp.int32, (_BN, 16), 1)
  g_ref[...] = jnp.where(lane < 8, nsf, ns)


def _k2p_body(cnt_ref, deg_ref, sqW_ref, sqb_ref, W0_ref, b_ref, ls_ref,
              lb_ref, W_ref, y_ref):
  ns, nd = _norms(deg_ref[...])
  cnt = cnt_ref[...]
  p = cnt[0][:, 0:1] + cnt[1][:, 0:1]
  q = cnt[0][:, 8:9] + cnt[1][:, 8:9]
  u = jnp.dot(sqW_ref[...], W0_ref[...], preferred_element_type=jnp.float32)
  v = jnp.dot(sqb_ref[...], W0_ref[...], preferred_element_type=jnp.float32)
  a = p * u + q * v
  t = a * nd + b_ref[...]
  mu = jnp.mean(t, axis=-1, keepdims=True)
  var = jnp.mean(jnp.square(t - mu), axis=-1, keepdims=True)
  t = (t - mu) * lax.rsqrt(var + 1e-5) * ls_ref[...] + lb_ref[...]
  x = jnp.maximum(t, 0.0)
  y = jnp.dot(x * ns, W_ref[...], preferred_element_type=jnp.float32)
  y_ref[0] = y[:, :32]
  y_ref[1] = y[:, 32:]


def _epilogue(agg_blk, nd, b, ls, lb):
  a = jnp.concatenate([agg_blk[0], agg_blk[1]], axis=-1)
  t = a * nd + b
  mu = jnp.mean(t, axis=-1, keepdims=True)
  var = jnp.mean(jnp.square(t - mu), axis=-1, keepdims=True)
  t = (t - mu) * lax.rsqrt(var + 1e-5) * ls + lb
  return jnp.maximum(t, 0.0)


def _k2_body(agg_ref, deg_ref, b_ref, ls_ref, lb_ref, W_ref, y_ref):
  ns, nd = _norms(deg_ref[...])
  x = _epilogue(agg_ref[...], nd, b_ref[...], ls_ref[...], lb_ref[...])
  y = jnp.dot(x * ns, W_ref[...], preferred_element_type=jnp.float32)
  y_ref[0] = y[:, :32]
  y_ref[1] = y[:, 32:]


def _k34_body(agg_ref, deg_ref, b_ref, ls_ref, lb_ref, cW_ref, cb_ref,
              y_ref):
  _, nd = _norms(deg_ref[...])
  x = _epilogue(agg_ref[...], nd, b_ref[...], ls_ref[...], lb_ref[...])
  xr = jnp.reshape(x, (_BN // L, L, R))
  acc = jnp.zeros((_BN // L, NCLS), jnp.float32) + cb_ref[...]
  for j in range(L):
    acc = acc + jnp.dot(xr[:, j, :], cW_ref[j],
                        preferred_element_type=jnp.float32)
  i = pl.program_id(0)
  y_ref[pl.ds(i * (_BN // L), _BN // L), :] = acc


def _full(shape):
  return pl.BlockSpec(shape, lambda i: tuple(0 for _ in shape))


def _rows3(shape):
  return pl.BlockSpec(shape, lambda i: (0, i, 0))


def _g1_stage(feats_col, deg2):
  return pl.pallas_call(
      _g1_body,
      grid=(N // _BN,),
      in_specs=[
          pl.BlockSpec((_BN, 1), lambda i: (i, 0)),
          _rows3((2, _BN, 16)),
      ],
      out_specs=pl.BlockSpec((_BN, 16), lambda i: (i, 0)),
      out_shape=jax.ShapeDtypeStruct((N, 16), jnp.float32),
  )(feats_col, deg2)


def _layer_matmul_p(cnt1, deg2, seq_W, seq_b, W0, b, ls, lb, W):
  return pl.pallas_call(
      _k2p_body,
      grid=(N // _BN,),
      in_specs=[
          _rows3((2, _BN, 16)),
          _rows3((2, _BN, 16)),
          _full((1, R)),
          _full((1, R)),
          _full((R, R)),
          _full((1, R)),
          _full((1, R)),
          _full((1, R)),
          _full((R, R)),
      ],
      out_specs=_rows3((2, _BN, 32)),
      out_shape=jax.ShapeDtypeStruct((2, N, 32), jnp.float32),
  )(cnt1, deg2, seq_W, seq_b, W0, b, ls, lb, W)


def _layer_matmul_mid(agg2, deg2, b, ls, lb, W):
  return pl.pallas_call(
      _k2_body,
      grid=(N // _BN,),
      in_specs=[
          _rows3((2, _BN, 32)),
          _rows3((2, _BN, 16)),
          _full((1, R)),
          _full((1, R)),
          _full((1, R)),
          _full((R, R)),
      ],
      out_specs=_rows3((2, _BN, 32)),
      out_shape=jax.ShapeDtypeStruct((2, N, 32), jnp.float32),
  )(agg2, deg2, b, ls, lb, W)


def _final_classifier(agg2, deg2, b, ls, lb, cls_W3, cls_b):
  return pl.pallas_call(
      _k34_body,
      grid=(N // _BN,),
      in_specs=[
          _rows3((2, _BN, 32)),
          _rows3((2, _BN, 16)),
          _full((1, R)),
          _full((1, R)),
          _full((1, R)),
          _full((L, R, NCLS)),
          _full((1, NCLS)),
      ],
      out_specs=pl.BlockSpec((N // L, NCLS), lambda i: (0, 0)),
      out_shape=jax.ShapeDtypeStruct((N // L, NCLS), jnp.float32),
  )(agg2, deg2, b, ls, lb, cls_W3, cls_b)


def kernel(feats, edge_index, seq_W, seq_b, W0, b0, W1, b1, W2, b2,
           ln_s0, ln_b0, ln_s1, ln_b1, ln_s2, ln_b2, cls_W, cls_b):
  src = edge_index[0]
  dst = edge_index[1]
  srcr = src.reshape(EROWS, 128)
  dstr = dst.reshape(EROWS, 128)
  npad = EROWS_P - EROWS
  lanes = lax.broadcasted_iota(jnp.int32, (npad, 128), 1)
  pad_src = lanes
  pad_trash = N + (lanes % (NP - N))
  srcr_p = jnp.concatenate([srcr, pad_src], axis=0)
  dstr_p = jnp.concatenate([dstr, pad_trash], axis=0)
  src2 = jnp.stack([srcr_p, srcr_p + N])
  hidx = jnp.stack([jnp.concatenate([srcr, pad_trash], axis=0), dstr_p])
  ones16 = jnp.ones((128, 16), jnp.float32)
  zeros16 = jnp.zeros((ROWS_PT, 16), jnp.float32)
  zeros32 = jnp.zeros((ROWS_PT, 32), jnp.float32)

  cnt = _degrees(hidx, ones16, zeros16)
  deg2 = cnt.reshape(2, NP, 16)

  feats_col = feats.reshape(N, 1)
  row = lambda v: v.reshape(1, R)

  g1 = _g1_stage(feats_col, deg2)
  cnt1 = _edge_agg16(g1, src2, dstr_p, zeros16).reshape(2, NP, 16)

  y = _layer_matmul_p(cnt1, deg2, seq_W, row(seq_b), W0, row(b0),
                      row(ln_s0), row(ln_b0), W1)
  agg2 = _edge_agg(y.reshape(2 * N, 32), src2, dstr_p,
                   zeros32).reshape(2, NP, 32)

  for (b, ls, lb, Wn) in ((b1, ln_s1, ln_b1, W2),):
    y = _layer_matmul_mid(agg2, deg2, row(b), row(ls), row(lb), Wn)
    agg2 = _edge_agg(y.reshape(2 * N, 32), src2, dstr_p,
                     zeros32).reshape(2, NP, 32)

  return _final_classifier(agg2, deg2, row(b2), row(ln_s2), row(ln_b2),
                           cls_W.reshape(L, R, NCLS),
                           jnp.reshape(cls_b, (1, NCLS)))
